# baseline (speedup 1.0000x reference)
import functools

import jax
import jax.numpy as jnp
from jax import lax
from jax.experimental import pallas as pl
from jax.experimental.pallas import tpu as pltpu
from jax.experimental.pallas import tpu_sc as plsc

NEG_SLOPE = 0.2
EPS_BN = 1e-5

_L = 16
_NSUB = 16
_NCORE = 2
_NW = _NSUB * _NCORE
_CH = 96


def _tc_pre(x, Wsat, bsat, Wnei, bnei, Wfa, Wfb, bfus, W1, asrc, adst):
    n, h = x.shape[0], W1.shape[0]

    def body(x_r, wsat_r, bsat_r, wnei_r, bnei_r, wfa_r, wfb_r, bfus_r,
             w1_r, as_r, ad_r, h0_r, z_r, zs_r, zd_r):
        xv = x_r[...]
        sat = jnp.maximum(xv @ wsat_r[...] + bsat_r[...], 0.0)
        nei = jnp.maximum(xv @ wnei_r[...] + bnei_r[...], 0.0)
        gl = sat @ wfa_r[...] + nei @ wfb_r[...] + bfus_r[...]
        gate = 1.0 / (1.0 + jnp.exp(-gl))
        h0 = gate * sat + (1.0 - gate) * nei
        z = h0 @ w1_r[...]
        h0_r[...] = h0
        z_r[...] = z
        zs_r[...] = z @ as_r[...]
        zd_r[...] = z @ ad_r[...]

    return pl.pallas_call(
        body,
        out_shape=(
            jax.ShapeDtypeStruct((n, h), jnp.float32),
            jax.ShapeDtypeStruct((n, h), jnp.float32),
            jax.ShapeDtypeStruct((n, 1), jnp.float32),
            jax.ShapeDtypeStruct((n, 1), jnp.float32),
        ),
    )(x, Wsat, bsat, Wnei, bnei, Wfa, Wfb, bfus, W1, asrc, adst)


def _tc_mid(num, den, h0, bc, g, be, W2, asrc, adst):
    n, h = h0.shape

    def body(num_r, den_r, h0_r, bc_r, g_r, be_r, w2_r, as_r, ad_r,
             h1_r, z_r, zs_r, zd_r):
        s = num_r[0, :n, :] + num_r[1, :n, :]
        d = den_r[0, :n, :] + den_r[1, :n, :]
        gat = s / (d + 1e-16) + bc_r[...]
        mu = jnp.mean(gat, axis=0, keepdims=True)
        var = jnp.mean((gat - mu) ** 2, axis=0, keepdims=True)
        bn = g_r[...] * (gat - mu) * lax.rsqrt(var + EPS_BN) + be_r[...]
        h1 = jnp.maximum(bn, 0.0) + h0_r[...]
        z = h1 @ w2_r[...]
        h1_r[...] = h1
        z_r[...] = z
        zs_r[...] = z @ as_r[...]
        zd_r[...] = z @ ad_r[...]

    return pl.pallas_call(
        body,
        out_shape=(
            jax.ShapeDtypeStruct((n, h), jnp.float32),
            jax.ShapeDtypeStruct((n, h), jnp.float32),
            jax.ShapeDtypeStruct((n, 1), jnp.float32),
            jax.ShapeDtypeStruct((n, 1), jnp.float32),
        ),
    )(num, den, h0, bc, g, be, W2, asrc, adst)


def _tc_post(num, den, h1, bc, g, be, Wf1, bf1, Wf2, bf2):
    n, h = h1.shape
    out_d = Wf2.shape[1]

    def body(num_r, den_r, h1_r, bc_r, g_r, be_r, wf1_r, bf1_r, wf2_r, bf2_r,
             out_r):
        s = num_r[0, :n, :] + num_r[1, :n, :]
        d = den_r[0, :n, :] + den_r[1, :n, :]
        gat = s / (d + 1e-16) + bc_r[...]
        mu = jnp.mean(gat, axis=0, keepdims=True)
        var = jnp.mean((gat - mu) ** 2, axis=0, keepdims=True)
        bn = g_r[...] * (gat - mu) * lax.rsqrt(var + EPS_BN) + be_r[...]
        hh = jnp.maximum(bn, 0.0) + h1_r[...]
        hh = jnp.maximum(hh @ wf1_r[...] + bf1_r[...], 0.0)
        out_r[...] = hh @ wf2_r[...] + bf2_r[...]

    return pl.pallas_call(
        body,
        out_shape=jax.ShapeDtypeStruct((n, out_d), jnp.float32),
    )(num, den, h1, bc, g, be, Wf1, bf1, Wf2, bf2)


def _sc_gat(z, zs_pad, zd_pad, src, dst, n_acc, e_pad):
    h = z.shape[1]
    epw = e_pad // _NW
    nch = epw // _CH
    rpt = n_acc // _NSUB
    nrb = 8
    bs = rpt // nrb
    mesh = plsc.VectorSubcoreMesh(core_axis_name="c", subcore_axis_name="s")

    @functools.partial(
        pl.kernel,
        out_type=(
            jax.ShapeDtypeStruct((_NCORE, n_acc, h), jnp.float32),
            jax.ShapeDtypeStruct((_NCORE, n_acc), jnp.float32),
        ),
        mesh=mesh,
        compiler_params=pltpu.CompilerParams(needs_layout_passes=False),
        scratch_types=[
            pltpu.VMEM((n_acc,), jnp.float32),
            pltpu.VMEM((n_acc,), jnp.float32),
            pltpu.VMEM((_CH,), jnp.int32),
            pltpu.VMEM((_CH,), jnp.int32),
            pltpu.VMEM((_CH,), jnp.int32),
            pltpu.VMEM((_CH,), jnp.int32),
            pltpu.VMEM((_CH,), jnp.int32),
            pltpu.VMEM((_CH,), jnp.int32),
            pltpu.VMEM((_CH,), jnp.float32),
            pltpu.VMEM((_CH,), jnp.float32),
            pltpu.VMEM((_CH, h), jnp.float32),
            pltpu.VMEM((_CH, h), jnp.float32),
            pltpu.VMEM_SHARED((n_acc, h), jnp.float32),
            pltpu.VMEM_SHARED((n_acc,), jnp.float32),
            pltpu.VMEM((rpt,), jnp.float32),
            pltpu.SemaphoreType.DMA,
            pltpu.SemaphoreType.DMA,
            pltpu.SemaphoreType.DMA,
            pltpu.SemaphoreType.DMA,
            pltpu.SemaphoreType.DMA,
            pltpu.SemaphoreType.DMA,
        ],
    )
    def k(z_hbm, zs_hbm, zd_hbm, src_hbm, dst_hbm, num_hbm, den_hbm,
          zs_v, zd_v, sidx_a, sidx_b, didx_a, didx_b, dsc_a, dsc_b, ee_a, ee_b,
          rows_a, rows_b, acc, den_sh, red_v, gsem_a, gsem_b, isem_a, isem_b,
          ssem_a, ssem_b):
        cid = lax.axis_index("c")
        sid = lax.axis_index("s")
        wid = cid * _NSUB + sid
        zeros16 = jnp.zeros((_L,), jnp.float32)
        rows = rows_a

        def zrow(i, _):
            for j in range(h // _L):
                rows[i, pl.ds(j * _L, _L)] = zeros16
            return 0
        lax.fori_loop(0, _CH, zrow, 0)
        for b in range(nrb):
            pltpu.sync_copy(rows.at[pl.ds(0, bs)],
                            acc.at[pl.ds(sid * rpt + b * bs, bs)])

        def zred(i, _):
            red_v[pl.ds(i * _L, _L)] = zeros16
            return 0
        lax.fori_loop(0, rpt // _L, zred, 0)
        pltpu.sync_copy(red_v, den_sh.at[pl.ds(sid * rpt, rpt)])

        pltpu.sync_copy(zs_hbm, zs_v)
        pltpu.sync_copy(zd_hbm, zd_v)
        plsc.subcore_barrier()

        ebase = wid * epw

        def chunk_off(c):
            return ebase + jnp.minimum(c, nch - 1) * _CH

        def half(c, bufs):
            (sidx_p, didx_p, dsc_p, ee_p, rows_p, gsem_p, isem_p, ssem_p,
             sidx_q, didx_q, dsc_q, ee_q, rows_q, gsem_q, isem_q, ssem_q) = bufs
            pltpu.make_async_copy(
                src_hbm.at[pl.ds(ebase, _CH)], sidx_q, isem_q).wait()
            pltpu.make_async_copy(
                dst_hbm.at[pl.ds(ebase, _CH)], didx_q, isem_q).wait()

            @pl.when(c >= 1)
            def _():
                pltpu.make_async_copy(
                    ee_q, den_sh.at[dsc_q], ssem_q).wait()
                pltpu.make_async_copy(
                    rows_q, acc.at[dsc_q], ssem_q).wait()

            pltpu.async_copy(z_hbm.at[sidx_q], rows_q, gsem_q)
            for i in range(_CH // _L):
                s16 = sidx_p[pl.ds(i * _L, _L)]
                d16 = didx_p[pl.ds(i * _L, _L)]
                dsc_p[pl.ds(i * _L, _L)] = d16
                v = plsc.load_gather(zs_v, [s16]) + plsc.load_gather(zd_v, [d16])
                v = jnp.where(v >= 0.0, v, v * NEG_SLOPE)
                ee_p[pl.ds(i * _L, _L)] = jnp.exp(v)
            pltpu.async_copy(ee_p, den_sh.at[dsc_p], ssem_p, add=True)
            pltpu.make_async_copy(z_hbm.at[sidx_p], rows_p, gsem_p).wait()
            pltpu.async_copy(
                src_hbm.at[pl.ds(chunk_off(c + 2), _CH)], sidx_p, isem_p)
            pltpu.async_copy(
                dst_hbm.at[pl.ds(chunk_off(c + 2), _CH)], didx_p, isem_p)

            @plsc.parallel_loop(0, _CH, step=1, unroll=8)
            def scale(r):
                ev = plsc.load_gather(ee_p, [jnp.full((_L,), r, jnp.int32)])
                for j in range(h // _L):
                    rows_p[r, pl.ds(j * _L, _L)] = (
                        rows_p[r, pl.ds(j * _L, _L)] * ev)

            pltpu.async_copy(rows_p, acc.at[dsc_p], ssem_p, add=True)

        bufs_a = (sidx_a, didx_a, dsc_a, ee_a, rows_a, gsem_a, isem_a, ssem_a,
                  sidx_b, didx_b, dsc_b, ee_b, rows_b, gsem_b, isem_b, ssem_b)
        bufs_b = bufs_a[8:] + bufs_a[:8]

        pltpu.sync_copy(src_hbm.at[pl.ds(ebase, _CH)], sidx_a)
        pltpu.sync_copy(dst_hbm.at[pl.ds(ebase, _CH)], didx_a)
        pltpu.async_copy(z_hbm.at[sidx_a], rows_a, gsem_a)
        pltpu.async_copy(src_hbm.at[pl.ds(ebase + _CH, _CH)], sidx_b, isem_b)
        pltpu.async_copy(dst_hbm.at[pl.ds(ebase + _CH, _CH)], didx_b, isem_b)

        def pair(i, _):
            half(2 * i, bufs_a)
            half(2 * i + 1, bufs_b)
            return 0
        lax.fori_loop(0, nch // 2, pair, 0)

        pltpu.make_async_copy(z_hbm.at[sidx_a], rows_a, gsem_a).wait()
        pltpu.make_async_copy(
            src_hbm.at[pl.ds(ebase, _CH)], sidx_b, isem_b).wait()
        pltpu.make_async_copy(
            dst_hbm.at[pl.ds(ebase, _CH)], didx_b, isem_b).wait()
        pltpu.make_async_copy(ee_b, den_sh.at[dsc_b], ssem_b).wait()
        pltpu.make_async_copy(rows_b, acc.at[dsc_b], ssem_b).wait()

        plsc.subcore_barrier()

        for b in range(nrb):
            r0 = sid * rpt + b * bs
            pltpu.sync_copy(acc.at[pl.ds(r0, bs)], rows.at[pl.ds(0, bs)])
            pltpu.sync_copy(rows.at[pl.ds(0, bs)], num_hbm.at[cid, pl.ds(r0, bs)])
        pltpu.sync_copy(den_sh.at[pl.ds(sid * rpt, rpt)], red_v)
        pltpu.sync_copy(red_v, den_hbm.at[cid, pl.ds(sid * rpt, rpt)])

    return k(z, zs_pad, zd_pad, src, dst)


def kernel(x, W_sat, b_sat, W_nei, b_nei, W_fus, b_fus, W1, a_src1, a_dst1,
           bc1, g1, be1, W2, a_src2, a_dst2, bc2, g2, be2, Wf1, bf1, Wf2, bf2,
           edge_index):
    n, d_in = x.shape
    sat_d = W_sat.shape[0]
    h = W_sat.shape[1]
    e = edge_index.shape[1]

    e_tot = e + n
    e_pad = -(-e_tot // (2 * _NW * _CH)) * (2 * _NW * _CH)
    n_acc = -(-(n + 1) // 1024) * 1024

    Wsat_f = jnp.concatenate(
        [W_sat, jnp.zeros((d_in - sat_d, h), jnp.float32)], axis=0)
    Wnei_f = jnp.concatenate(
        [jnp.zeros((sat_d, h), jnp.float32), W_nei], axis=0)
    Wfa, Wfb = W_fus[:h], W_fus[h:]
    r2 = lambda v: v.reshape(1, -1)
    c2 = lambda v: v.reshape(-1, 1)

    si = jnp.arange(n, dtype=edge_index.dtype)
    src = jnp.concatenate(
        [edge_index[0], si, jnp.zeros((e_pad - e_tot,), edge_index.dtype)])
    dst = jnp.concatenate(
        [edge_index[1], si, jnp.full((e_pad - e_tot,), n, edge_index.dtype)])

    padv = lambda v: jnp.pad(v[:, 0], (0, n_acc - n))

    h0, z1, zs1, zd1 = _tc_pre(x, Wsat_f, r2(b_sat), Wnei_f, r2(b_nei),
                               Wfa, Wfb, r2(b_fus), W1, c2(a_src1), c2(a_dst1))
    num1, den1 = _sc_gat(z1, padv(zs1), padv(zd1), src, dst, n_acc, e_pad)
    h1, z2, zs2, zd2 = _tc_mid(num1, den1.reshape(_NCORE, n_acc, 1), h0,
                               r2(bc1), r2(g1), r2(be1), W2,
                               c2(a_src2), c2(a_dst2))
    num2, den2 = _sc_gat(z2, padv(zs2), padv(zd2), src, dst, n_acc, e_pad)
    return _tc_post(num2, den2.reshape(_NCORE, n_acc, 1), h1,
                    r2(bc2), r2(g2), r2(be2), Wf1, r2(bf1), Wf2, r2(bf2))


# speedup vs baseline: 1.3234x; 1.3234x over previous
import functools

import jax
import jax.numpy as jnp
from jax import lax
from jax.experimental import pallas as pl
from jax.experimental.pallas import tpu as pltpu
from jax.experimental.pallas import tpu_sc as plsc

NEG_SLOPE = 0.2
EPS_BN = 1e-5

_L = 16
_NSUB = 16
_NCORE = 2
_NW = _NSUB * _NCORE
_CH = 96


def _tc_pre(x, Wsat, bsat, Wnei, bnei, Wfa, Wfb, bfus, W1, asrc, adst):
    n, h = x.shape[0], W1.shape[0]

    def body(x_r, wsat_r, bsat_r, wnei_r, bnei_r, wfa_r, wfb_r, bfus_r,
             w1_r, as_r, ad_r, h0_r, z_r, zs_r, zd_r):
        xv = x_r[...]
        sat = jnp.maximum(xv @ wsat_r[...] + bsat_r[...], 0.0)
        nei = jnp.maximum(xv @ wnei_r[...] + bnei_r[...], 0.0)
        gl = sat @ wfa_r[...] + nei @ wfb_r[...] + bfus_r[...]
        gate = 1.0 / (1.0 + jnp.exp(-gl))
        h0 = gate * sat + (1.0 - gate) * nei
        z = h0 @ w1_r[...]
        h0_r[...] = h0
        z_r[...] = z
        zs_r[...] = z @ as_r[...]
        zd_r[...] = z @ ad_r[...]

    return pl.pallas_call(
        body,
        out_shape=(
            jax.ShapeDtypeStruct((n, h), jnp.float32),
            jax.ShapeDtypeStruct((n, h), jnp.float32),
            jax.ShapeDtypeStruct((n, 1), jnp.float32),
            jax.ShapeDtypeStruct((n, 1), jnp.float32),
        ),
    )(x, Wsat, bsat, Wnei, bnei, Wfa, Wfb, bfus, W1, asrc, adst)


def _tc_mid(num, den, h0, bc, g, be, W2, asrc, adst):
    n, h = h0.shape

    def body(num_r, den_r, h0_r, bc_r, g_r, be_r, w2_r, as_r, ad_r,
             h1_r, z_r, zs_r, zd_r):
        s = num_r[0, :n, :] + num_r[1, :n, :]
        d = den_r[0, :n, :] + den_r[1, :n, :]
        gat = s / (d + 1e-16) + bc_r[...]
        mu = jnp.mean(gat, axis=0, keepdims=True)
        var = jnp.mean((gat - mu) ** 2, axis=0, keepdims=True)
        bn = g_r[...] * (gat - mu) * lax.rsqrt(var + EPS_BN) + be_r[...]
        h1 = jnp.maximum(bn, 0.0) + h0_r[...]
        z = h1 @ w2_r[...]
        h1_r[...] = h1
        z_r[...] = z
        zs_r[...] = z @ as_r[...]
        zd_r[...] = z @ ad_r[...]

    return pl.pallas_call(
        body,
        out_shape=(
            jax.ShapeDtypeStruct((n, h), jnp.float32),
            jax.ShapeDtypeStruct((n, h), jnp.float32),
            jax.ShapeDtypeStruct((n, 1), jnp.float32),
            jax.ShapeDtypeStruct((n, 1), jnp.float32),
        ),
    )(num, den, h0, bc, g, be, W2, asrc, adst)


def _tc_post(num, den, h1, bc, g, be, Wf1, bf1, Wf2, bf2):
    n, h = h1.shape
    out_d = Wf2.shape[1]

    def body(num_r, den_r, h1_r, bc_r, g_r, be_r, wf1_r, bf1_r, wf2_r, bf2_r,
             out_r):
        s = num_r[0, :n, :] + num_r[1, :n, :]
        d = den_r[0, :n, :] + den_r[1, :n, :]
        gat = s / (d + 1e-16) + bc_r[...]
        mu = jnp.mean(gat, axis=0, keepdims=True)
        var = jnp.mean((gat - mu) ** 2, axis=0, keepdims=True)
        bn = g_r[...] * (gat - mu) * lax.rsqrt(var + EPS_BN) + be_r[...]
        hh = jnp.maximum(bn, 0.0) + h1_r[...]
        hh = jnp.maximum(hh @ wf1_r[...] + bf1_r[...], 0.0)
        out_r[...] = hh @ wf2_r[...] + bf2_r[...]

    return pl.pallas_call(
        body,
        out_shape=jax.ShapeDtypeStruct((n, out_d), jnp.float32),
    )(num, den, h1, bc, g, be, Wf1, bf1, Wf2, bf2)


def _sc_gat(z, zs_pad, zd_pad, src, dst, n_acc, e_pad):
    h = z.shape[1]
    epw = e_pad // _NW
    nch = epw // _CH
    rpt = n_acc // _NSUB
    nrb = 8
    bs = rpt // nrb
    mesh = plsc.VectorSubcoreMesh(core_axis_name="c", subcore_axis_name="s")

    @functools.partial(
        pl.kernel,
        out_type=(
            jax.ShapeDtypeStruct((_NCORE, n_acc, h), jnp.float32),
            jax.ShapeDtypeStruct((_NCORE, n_acc), jnp.float32),
        ),
        mesh=mesh,
        compiler_params=pltpu.CompilerParams(needs_layout_passes=False),
        scratch_types=[
            pltpu.VMEM((n_acc,), jnp.float32),
            pltpu.VMEM((n_acc,), jnp.float32),
            pltpu.VMEM((_CH,), jnp.int32),
            pltpu.VMEM((_CH,), jnp.int32),
            pltpu.VMEM((_CH,), jnp.int32),
            pltpu.VMEM((_CH,), jnp.int32),
            pltpu.VMEM((_CH,), jnp.int32),
            pltpu.VMEM((_CH,), jnp.int32),
            pltpu.VMEM((_CH,), jnp.float32),
            pltpu.VMEM((_CH,), jnp.float32),
            pltpu.VMEM((_CH, h), jnp.float32),
            pltpu.VMEM((_CH, h), jnp.float32),
            pltpu.VMEM_SHARED((n_acc, h), jnp.float32),
            pltpu.VMEM_SHARED((n_acc,), jnp.float32),
            pltpu.VMEM((rpt,), jnp.float32),
            pltpu.SemaphoreType.DMA,
            pltpu.SemaphoreType.DMA,
            pltpu.SemaphoreType.DMA,
            pltpu.SemaphoreType.DMA,
            pltpu.SemaphoreType.DMA,
            pltpu.SemaphoreType.DMA,
        ],
    )
    def k(z_hbm, zs_hbm, zd_hbm, src_hbm, dst_hbm, num_hbm, den_hbm,
          zs_v, zd_v, sidx_a, sidx_b, didx_a, didx_b, dsc_a, dsc_b, ee_a, ee_b,
          rows_a, rows_b, acc, den_sh, red_v, gsem_a, gsem_b, isem_a, isem_b,
          ssem_a, ssem_b):
        cid = lax.axis_index("c")
        sid = lax.axis_index("s")
        wid = cid * _NSUB + sid
        zeros16 = jnp.zeros((_L,), jnp.float32)
        rows = rows_a

        def zrow(i, _):
            for j in range(h // _L):
                rows[i, pl.ds(j * _L, _L)] = zeros16
            return 0
        lax.fori_loop(0, _CH, zrow, 0)
        for b in range(nrb):
            pltpu.sync_copy(rows.at[pl.ds(0, bs)],
                            acc.at[pl.ds(sid * rpt + b * bs, bs)])

        def zred(i, _):
            red_v[pl.ds(i * _L, _L)] = zeros16
            return 0
        lax.fori_loop(0, rpt // _L, zred, 0)
        pltpu.sync_copy(red_v, den_sh.at[pl.ds(sid * rpt, rpt)])

        pltpu.sync_copy(zs_hbm, zs_v)
        pltpu.sync_copy(zd_hbm, zd_v)
        plsc.subcore_barrier()

        ebase = wid * epw

        def chunk_off(c):
            return ebase + jnp.minimum(c, nch - 1) * _CH

        def half(c, bufs):
            (sidx_p, didx_p, dsc_p, ee_p, rows_p, gsem_p, isem_p, ssem_p,
             sidx_q, didx_q, dsc_q, ee_q, rows_q, gsem_q, isem_q, ssem_q) = bufs
            pltpu.make_async_copy(
                src_hbm.at[pl.ds(ebase, _CH)], sidx_q, isem_q).wait()
            pltpu.make_async_copy(
                dst_hbm.at[pl.ds(ebase, _CH)], didx_q, isem_q).wait()

            @pl.when(c >= 1)
            def _():
                pltpu.make_async_copy(
                    ee_q, den_sh.at[dsc_q], ssem_q).wait()
                pltpu.make_async_copy(
                    rows_q, acc.at[dsc_q], ssem_q).wait()

            pltpu.async_copy(z_hbm.at[sidx_q], rows_q, gsem_q)
            for i in range(_CH // _L):
                s16 = sidx_p[pl.ds(i * _L, _L)]
                d16 = didx_p[pl.ds(i * _L, _L)]
                dsc_p[pl.ds(i * _L, _L)] = d16
                v = plsc.load_gather(zs_v, [s16]) + plsc.load_gather(zd_v, [d16])
                v = jnp.where(v >= 0.0, v, v * NEG_SLOPE)
                ee_p[pl.ds(i * _L, _L)] = jnp.exp(v)
            pltpu.async_copy(ee_p, den_sh.at[dsc_p], ssem_p, add=True)
            pltpu.make_async_copy(z_hbm.at[sidx_p], rows_p, gsem_p).wait()
            pltpu.async_copy(
                src_hbm.at[pl.ds(chunk_off(c + 2), _CH)], sidx_p, isem_p)
            pltpu.async_copy(
                dst_hbm.at[pl.ds(chunk_off(c + 2), _CH)], didx_p, isem_p)

            @plsc.parallel_loop(0, _CH, step=1, unroll=8)
            def scale(r):
                ev = plsc.load_gather(ee_p, [jnp.full((_L,), r, jnp.int32)])
                for j in range(h // _L):
                    rows_p[r, pl.ds(j * _L, _L)] = (
                        rows_p[r, pl.ds(j * _L, _L)] * ev)

            pltpu.async_copy(rows_p, acc.at[dsc_p], ssem_p, add=True)

        bufs_a = (sidx_a, didx_a, dsc_a, ee_a, rows_a, gsem_a, isem_a, ssem_a,
                  sidx_b, didx_b, dsc_b, ee_b, rows_b, gsem_b, isem_b, ssem_b)
        bufs_b = bufs_a[8:] + bufs_a[:8]

        pltpu.sync_copy(src_hbm.at[pl.ds(ebase, _CH)], sidx_a)
        pltpu.sync_copy(dst_hbm.at[pl.ds(ebase, _CH)], didx_a)
        pltpu.async_copy(z_hbm.at[sidx_a], rows_a, gsem_a)
        pltpu.async_copy(src_hbm.at[pl.ds(ebase + _CH, _CH)], sidx_b, isem_b)
        pltpu.async_copy(dst_hbm.at[pl.ds(ebase + _CH, _CH)], didx_b, isem_b)

        def pair(i, _):
            half(2 * i, bufs_a)
            half(2 * i + 1, bufs_b)
            return 0
        lax.fori_loop(0, nch // 2, pair, 0)

        pltpu.make_async_copy(z_hbm.at[sidx_a], rows_a, gsem_a).wait()
        pltpu.make_async_copy(
            src_hbm.at[pl.ds(ebase, _CH)], sidx_b, isem_b).wait()
        pltpu.make_async_copy(
            dst_hbm.at[pl.ds(ebase, _CH)], didx_b, isem_b).wait()
        pltpu.make_async_copy(ee_b, den_sh.at[dsc_b], ssem_b).wait()
        pltpu.make_async_copy(rows_b, acc.at[dsc_b], ssem_b).wait()

        plsc.subcore_barrier()

        pltpu.sync_copy(acc.at[pl.ds(sid * rpt, rpt)],
                        num_hbm.at[cid, pl.ds(sid * rpt, rpt)])
        pltpu.sync_copy(den_sh.at[pl.ds(sid * rpt, rpt)], red_v)
        pltpu.sync_copy(red_v, den_hbm.at[cid, pl.ds(sid * rpt, rpt)])

    return k(z, zs_pad, zd_pad, src, dst)


def kernel(x, W_sat, b_sat, W_nei, b_nei, W_fus, b_fus, W1, a_src1, a_dst1,
           bc1, g1, be1, W2, a_src2, a_dst2, bc2, g2, be2, Wf1, bf1, Wf2, bf2,
           edge_index):
    n, d_in = x.shape
    sat_d = W_sat.shape[0]
    h = W_sat.shape[1]
    e = edge_index.shape[1]

    e_tot = e + n
    e_pad = -(-e_tot // (2 * _NW * _CH)) * (2 * _NW * _CH)
    n_acc = -(-(n + 1) // 1024) * 1024

    Wsat_f = jnp.concatenate(
        [W_sat, jnp.zeros((d_in - sat_d, h), jnp.float32)], axis=0)
    Wnei_f = jnp.concatenate(
        [jnp.zeros((sat_d, h), jnp.float32), W_nei], axis=0)
    Wfa, Wfb = W_fus[:h], W_fus[h:]
    r2 = lambda v: v.reshape(1, -1)
    c2 = lambda v: v.reshape(-1, 1)

    si = jnp.arange(n, dtype=edge_index.dtype)
    pi = jnp.arange(e_pad - e_tot, dtype=edge_index.dtype)
    src = jnp.concatenate([edge_index[0], si, pi % n])
    dst = jnp.concatenate([edge_index[1], si, n + pi % (n_acc - n)])

    padv = lambda v: jnp.pad(v[:, 0], (0, n_acc - n))

    h0, z1, zs1, zd1 = _tc_pre(x, Wsat_f, r2(b_sat), Wnei_f, r2(b_nei),
                               Wfa, Wfb, r2(b_fus), W1, c2(a_src1), c2(a_dst1))
    num1, den1 = _sc_gat(z1, padv(zs1), padv(zd1), src, dst, n_acc, e_pad)
    h1, z2, zs2, zd2 = _tc_mid(num1, den1.reshape(_NCORE, n_acc, 1), h0,
                               r2(bc1), r2(g1), r2(be1), W2,
                               c2(a_src2), c2(a_dst2))
    num2, den2 = _sc_gat(z2, padv(zs2), padv(zd2), src, dst, n_acc, e_pad)
    return _tc_post(num2, den2.reshape(_NCORE, n_acc, 1), h1,
                    r2(bc2), r2(g2), r2(be2), Wf1, r2(bf1), Wf2, r2(bf2))
